# baseline (speedup 1.0000x reference)
import jax
import jax.numpy as jnp
from jax import lax
from jax.experimental import pallas as pl
from jax.experimental.pallas import tpu as pltpu
from jax.experimental.pallas import tpu_sc as plsc

B, H, R, W2, C, K = 2, 16, 64, 16, 128, 8
NBH = B * H
ROWS_PER_W = R * K
NC, NS = 2, 16
LANES = 16
GJ = 16
NOUT = ROWS_PER_W // GJ
RROWS = R * W2 // 2


def _kv_gather_body(idx_hbm, w_hbm, table_hbm, out_hbm,
                    region_v, idx_v, w_v, buf, ssems):
    wid = lax.axis_index("s") * NC + lax.axis_index("c")
    out_base = wid * ROWS_PER_W

    pltpu.sync_copy(table_hbm.at[pl.ds(wid * RROWS, RROWS)], region_v)
    pltpu.sync_copy(idx_hbm.at[pl.ds(wid * 4, 4)], idx_v)
    pltpu.sync_copy(w_hbm.at[pl.ds(wid * 64, 64)], w_v)

    def row_dma(lr, j):
        return pltpu.make_async_copy(
            buf.at[pl.ds(j, 1)],
            out_hbm.at[pl.ds(out_base + lr, 1)], ssems[j])

    def outer(o, _):
        idxv = idx_v[o >> 3, pl.ds(pl.multiple_of((o & 7) * LANES, LANES),
                                   LANES)]
        for j in range(GJ):
            lr = o * GJ + j

            @pl.when(o > 0)
            def _():
                row_dma(lr - GJ, j).wait()

            wv = w_v[lr >> 3, pl.ds((j & 7) * LANES, LANES)]
            rbase = idxv[j] * (W2 // 2)

            def sbody(s2, _):
                for c8 in range(8):
                    xi = region_v[rbase + s2, pl.ds(c8 * LANES, LANES)]
                    a = ((xi << 16) >> 16).astype(jnp.float32)
                    b = (xi >> 16).astype(jnp.float32)
                    s = 2 * s2 + c8 // 4
                    col = (c8 % 4) * 32
                    buf[j, s, pl.ds(col, LANES)] = a * wv
                    buf[j, s, pl.ds(col + LANES, LANES)] = b * wv
                return 0

            lax.fori_loop(0, W2 // 2, sbody, 0, unroll=2)
            row_dma(lr, j).start()
        return 0

    lax.fori_loop(0, NOUT, outer, 0)

    for j in range(GJ):
        row_dma((NOUT - 1) * GJ + j, j).wait()


def _body(idx_hbm, w_hbm, table_hbm, out_hbm, region_v, idx_v, w_v, buf,
          s0, s1, s2, s3, s4, s5, s6, s7, s8, s9, s10, s11, s12, s13,
          s14, s15):
    _kv_gather_body(idx_hbm, w_hbm, table_hbm, out_hbm, region_v, idx_v,
                    w_v, buf,
                    (s0, s1, s2, s3, s4, s5, s6, s7, s8, s9, s10, s11,
                     s12, s13, s14, s15))


@jax.jit
def _kv_gather(idx, w, table):
    mesh = plsc.VectorSubcoreMesh(core_axis_name="c", subcore_axis_name="s")
    return pl.kernel(
        _body,
        out_type=jax.ShapeDtypeStruct((NBH * ROWS_PER_W, W2, C), jnp.float32),
        mesh=mesh,
        scratch_types=[
            pltpu.VMEM((RROWS, C), jnp.int32),
            pltpu.VMEM((4, C), jnp.int32),
            pltpu.VMEM((64, C), jnp.float32),
            pltpu.VMEM((GJ, W2, C), jnp.float32),
        ] + [pltpu.SemaphoreType.DMA] * GJ,
    )(idx, w, table)


def kernel(r_idx, r_weight, kv):
    idx = r_idx.reshape(NBH * 4, C)
    maxabs = jnp.max(jnp.abs(kv))
    r = 32766.0 / jnp.maximum(maxabs, jnp.float32(1e-30))
    w = r_weight.reshape(NBH * ROWS_PER_W, 1) / r
    w = jnp.broadcast_to(w, (NBH * ROWS_PER_W, LANES)).reshape(NBH * 64, C)
    q = jnp.rint(kv * r).astype(jnp.int16)
    table = (q.reshape(NBH * R, C // 2, 2, LANES)
             .swapaxes(2, 3)
             .reshape(-1, 2))
    table = jax.lax.bitcast_convert_type(table, jnp.int32)
    table = table.reshape(NBH * R * W2 // 2, C)
    out = _kv_gather(idx, w, table)
    return out.reshape(B, H, R, K, W2, C)


# speedup vs baseline: 1.3710x; 1.3710x over previous
import jax
import jax.numpy as jnp
from jax import lax
from jax.experimental import pallas as pl
from jax.experimental.pallas import tpu as pltpu
from jax.experimental.pallas import tpu_sc as plsc

B, H, R, W2, C, K = 2, 16, 64, 16, 128, 8
NBH = B * H
ROWS_PER_W = R * K
NC, NS = 2, 16
LANES = 16
GJ = 16
NOUT = ROWS_PER_W // GJ
RROWS = R * W2 // 2


def _kv_gather_body(idx_hbm, w_hbm, table_hbm, out_hbm,
                    region_v, idx_v, w_v, buf, ssems):
    wid = lax.axis_index("s") * NC + lax.axis_index("c")
    out_base = wid * ROWS_PER_W

    pltpu.sync_copy(table_hbm.at[pl.ds(wid * RROWS, RROWS)], region_v)
    pltpu.sync_copy(idx_hbm.at[pl.ds(wid * 4, 4)], idx_v)
    pltpu.sync_copy(w_hbm.at[pl.ds(wid * 64, 64)], w_v)

    def row_dma(lr, j):
        return pltpu.make_async_copy(
            buf.at[pl.ds(j, 1)],
            out_hbm.at[pl.ds(out_base + lr, 1)], ssems[j])

    def outer(o, _):
        idxv = idx_v[o >> 3, pl.ds(pl.multiple_of((o & 7) * LANES, LANES),
                                   LANES)]
        for j in range(GJ):
            lr = o * GJ + j

            @pl.when(o > 0)
            def _():
                row_dma(lr - GJ, j).wait()

            wv = w_v[lr >> 3, pl.ds((j & 7) * LANES, LANES)]
            rbase = idxv[j] * (W2 // 2)

            @plsc.parallel_loop(0, W2 // 2, unroll=2)
            def _(s2):
                for c8 in range(8):
                    xi = region_v[rbase + s2, pl.ds(c8 * LANES, LANES)]
                    a = ((xi << 16) >> 16).astype(jnp.float32)
                    b = (xi >> 16).astype(jnp.float32)
                    s = 2 * s2 + c8 // 4
                    col = (c8 % 4) * 32
                    buf[j, s, pl.ds(col, LANES)] = a * wv
                    buf[j, s, pl.ds(col + LANES, LANES)] = b * wv
            row_dma(lr, j).start()
        return 0

    lax.fori_loop(0, NOUT, outer, 0)

    for j in range(GJ):
        row_dma((NOUT - 1) * GJ + j, j).wait()


def _body(idx_hbm, w_hbm, table_hbm, out_hbm, region_v, idx_v, w_v, buf,
          s0, s1, s2, s3, s4, s5, s6, s7, s8, s9, s10, s11, s12, s13,
          s14, s15):
    _kv_gather_body(idx_hbm, w_hbm, table_hbm, out_hbm, region_v, idx_v,
                    w_v, buf,
                    (s0, s1, s2, s3, s4, s5, s6, s7, s8, s9, s10, s11,
                     s12, s13, s14, s15))


@jax.jit
def _kv_gather(idx, w, table):
    mesh = plsc.VectorSubcoreMesh(core_axis_name="c", subcore_axis_name="s")
    return pl.kernel(
        _body,
        out_type=jax.ShapeDtypeStruct((NBH * ROWS_PER_W, W2, C), jnp.float32),
        mesh=mesh,
        scratch_types=[
            pltpu.VMEM((RROWS, C), jnp.int32),
            pltpu.VMEM((4, C), jnp.int32),
            pltpu.VMEM((64, C), jnp.float32),
            pltpu.VMEM((GJ, W2, C), jnp.float32),
        ] + [pltpu.SemaphoreType.DMA] * GJ,
    )(idx, w, table)


def kernel(r_idx, r_weight, kv):
    idx = r_idx.reshape(NBH * 4, C)
    maxabs = jnp.max(jnp.abs(kv))
    r = 32766.0 / jnp.maximum(maxabs, jnp.float32(1e-30))
    w = r_weight.reshape(NBH * ROWS_PER_W, 1) / r
    w = jnp.broadcast_to(w, (NBH * ROWS_PER_W, LANES)).reshape(NBH * 64, C)
    q = jnp.rint(kv * r).astype(jnp.int16)
    table = (q.reshape(NBH * R, C // 2, 2, LANES)
             .swapaxes(2, 3)
             .reshape(-1, 2))
    table = jax.lax.bitcast_convert_type(table, jnp.int32)
    table = table.reshape(NBH * R * W2 // 2, C)
    out = _kv_gather(idx, w, table)
    return out.reshape(B, H, R, K, W2, C)


# speedup vs baseline: 4.0229x; 2.9343x over previous
import jax
import jax.numpy as jnp
from jax import lax
from jax.experimental import pallas as pl
from jax.experimental.pallas import tpu as pltpu
from jax.experimental.pallas import tpu_sc as plsc

B, H, R, W2, C, K = 2, 16, 64, 16, 128, 8
NBH = B * H
ROWS_PER_W = R * K
NC, NS = 2, 16
LANES = 16
G = 8
NBUF = 4
SLOTS = 4
NWAVE = 16 // SLOTS
QS = NS // SLOTS
PART_ROWS = ROWS_PER_W // QS
NCHUNK = PART_ROWS // G
LOAD_ROWS = R // QS
IDXR = PART_ROWS // C
WR = PART_ROWS * LANES // C


def _scale_rows(buf, w_v, row0):
    for i in range(G):
        row = row0 + i
        wv = w_v[row >> 3, pl.ds(pl.multiple_of((row & 7) * LANES, LANES),
                                 LANES)]

        def body(s, _):
            for cj in range(C // LANES):
                sl = pl.ds(cj * LANES, LANES)

                buf[i, s, sl] = buf[i, s, sl] * wv
            return 0

        lax.fori_loop(0, W2, body, 0)


def _kv_gather_body(idx_hbm, w_hbm, table_hbm, out_hbm,
                    region_sh, idx_v, w_v, bufs, gsems, ssems):
    sc = lax.axis_index("c")
    tile = lax.axis_index("s")
    slot = tile % SLOTS
    part = tile // SLOTS

    for w in range(NWAVE):
        bh = sc * (NWAVE * SLOTS) + w * SLOTS + slot

        plsc.subcore_barrier()

        pltpu.sync_copy(
            table_hbm.at[pl.ds(bh * R + part * LOAD_ROWS, LOAD_ROWS)],
            region_sh.at[pl.ds(slot * R + part * LOAD_ROWS, LOAD_ROWS)])
        plsc.subcore_barrier()

        pltpu.sync_copy(idx_hbm.at[pl.ds(bh * 4 + part * IDXR, IDXR)], idx_v)
        pltpu.sync_copy(w_hbm.at[pl.ds(bh * 64 + part * WR, WR)], w_v)

        off = slot * R
        for r in range(IDXR):
            for t in range(C // LANES):
                sl = pl.ds(t * LANES, LANES)
                idx_v[r, sl] = idx_v[r, sl] + off

        out_base = bh * ROWS_PER_W + part * PART_ROWS

        def gather(g, b):
            src = region_sh.at[idx_v.at[g // (C // G),
                                        pl.ds((g % (C // G)) * G, G)]]
            return pltpu.make_async_copy(src, bufs[b], gsems[b])

        def scatter(g, b):
            dst = out_hbm.at[pl.ds(out_base + g * G, G)]
            return pltpu.make_async_copy(bufs[b], dst, ssems[b])

        gather(0, 0).start()
        gather(1, 1).start()

        def outer(o, _):
            for bpos in range(NBUF):
                g = o * NBUF + bpos
                gather(g, bpos).wait()
                _scale_rows(bufs[bpos], w_v, g * G)
                scatter(g, bpos).start()
                nxt = g + 2
                bn = (bpos + 2) % NBUF
                prev = g - 2

                @pl.when(nxt < NCHUNK)
                def _():
                    @pl.when(prev >= 0)
                    def _():
                        scatter(prev, bn).wait()

                    gather(nxt, bn).start()
            return 0

        lax.fori_loop(0, NCHUNK // NBUF, outer, 0)

        scatter(NCHUNK - 2, (NCHUNK - 2) % NBUF).wait()
        scatter(NCHUNK - 1, (NCHUNK - 1) % NBUF).wait()


def _body(idx_hbm, w_hbm, table_hbm, out_hbm,
          region_sh, idx_v, w_v, b0, b1, b2, b3, gs0, gs1, gs2, gs3,
          ss0, ss1, ss2, ss3):
    _kv_gather_body(idx_hbm, w_hbm, table_hbm, out_hbm, region_sh, idx_v,
                    w_v, (b0, b1, b2, b3), (gs0, gs1, gs2, gs3),
                    (ss0, ss1, ss2, ss3))


@jax.jit
def _kv_gather(idx, w, table):
    mesh = plsc.VectorSubcoreMesh(core_axis_name="c", subcore_axis_name="s")
    return pl.kernel(
        _body,
        out_type=jax.ShapeDtypeStruct((NBH * ROWS_PER_W, W2, C), jnp.float32),
        mesh=mesh,
        scratch_types=[
            pltpu.VMEM_SHARED((SLOTS * R, W2, C), jnp.float32),
            pltpu.VMEM((IDXR, C), jnp.int32),
            pltpu.VMEM((WR, C), jnp.float32),
            pltpu.VMEM((G, W2, C), jnp.float32),
            pltpu.VMEM((G, W2, C), jnp.float32),
            pltpu.VMEM((G, W2, C), jnp.float32),
            pltpu.VMEM((G, W2, C), jnp.float32),
            pltpu.SemaphoreType.DMA,
            pltpu.SemaphoreType.DMA,
            pltpu.SemaphoreType.DMA,
            pltpu.SemaphoreType.DMA,
            pltpu.SemaphoreType.DMA,
            pltpu.SemaphoreType.DMA,
            pltpu.SemaphoreType.DMA,
            pltpu.SemaphoreType.DMA,
        ],
    )(idx, w, table)


def kernel(r_idx, r_weight, kv):
    idx = r_idx.reshape(NBH * 4, C)
    w = jnp.broadcast_to(r_weight.reshape(NBH * ROWS_PER_W, 1),
                         (NBH * ROWS_PER_W, LANES))
    w = w.reshape(NBH * 64, C)
    table = kv.reshape(NBH * R, W2, C)
    out = _kv_gather(idx, w, table)
    return out.reshape(B, H, R, K, W2, C)
